# baseline (speedup 1.0000x reference)
import jax
import jax.numpy as jnp
from jax import lax
from jax.experimental import pallas as pl
from jax.experimental.pallas import tpu as pltpu
from jax.experimental.pallas import tpu_sc as plsc

NUM_CORES = 2
NUM_SUBCORES = 16
LANES = 16
NW = NUM_CORES * NUM_SUBCORES

BATCH = 16384
EMBED_DIM = 64
B_PER_W = BATCH // NW
WIN = 128
NWIN_TOT = 7813
WPW = 245
CAP = 16
RING = 6
FLUSH = 128
REGION = 1024
DUMMY = BATCH
IDBLK = 1024
LAST_START = (NWIN_TOT - 1) * WIN
NSTAGE = NW * REGION


def _one_table(ids_hbm, tab_hbm, rows_hbm, bl_hbm, wid, lane,
               ids_v, tmp_id, tmp_b, bk_id, bk_b, bk_cnt,
               ring, rowstage, blist_v, sem_ring, sem_st):
    wlo = wid * WPW

    z = jnp.zeros((LANES,), jnp.int32)
    for i in range(0, WPW + LANES, LANES):
        bk_cnt[pl.ds(i, LANES)] = z

    dv = jnp.full((LANES,), DUMMY, jnp.int32)
    for r in range(REGION // WIN):
        for i in range(0, WIN, LANES):
            blist_v[r, pl.ds(i, LANES)] = dv

    def blk_body(blk, _):
        pltpu.sync_copy(ids_hbm.at[pl.ds(blk * IDBLK, IDBLK)], ids_v)

        def v_body(v, _):
            ids16 = ids_v[pl.ds(v * LANES, LANES)]
            m = lax.div(ids16 >> 7, WPW) == wid
            n = plsc.all_reduce_population_count(m)[0]

            @pl.when(n > 0)
            def _binned():
                plsc.store_compressed(tmp_id.at[pl.ds(0, LANES)], ids16,
                                      mask=m)
                b16 = blk * IDBLK + v * LANES + lane
                plsc.store_compressed(tmp_b.at[pl.ds(0, LANES)], b16, mask=m)

                def j_body(j, _):
                    idj = tmp_id[pl.ds(j, LANES)][0]
                    bj = tmp_b[pl.ds(j, LANES)][0]
                    k = (idj >> 7) - wlo
                    c = bk_cnt[pl.ds(k, LANES)][0]
                    sel = lane == 0
                    kc = jnp.full((LANES,), k * CAP + c, jnp.int32)
                    plsc.store_scatter(bk_id, [kc],
                                       jnp.full((LANES,), idj, jnp.int32),
                                       mask=sel)
                    plsc.store_scatter(bk_b, [kc],
                                       jnp.full((LANES,), bj, jnp.int32),
                                       mask=sel)
                    plsc.store_scatter(bk_cnt,
                                       [jnp.full((LANES,), k, jnp.int32)],
                                       jnp.full((LANES,), c + 1, jnp.int32),
                                       mask=sel)
                    return 0

                lax.fori_loop(0, n, j_body, 0)

            return 0

        lax.fori_loop(0, IDBLK // LANES, v_body, 0)
        return 0

    lax.fori_loop(0, BATCH // IDBLK, blk_body, 0)

    def issue(k):
        start = pl.multiple_of(
            jnp.minimum((wlo + k) * WIN, LAST_START), WIN)
        return pltpu.async_copy(tab_hbm.at[:, pl.ds(start, WIN)],
                                ring.at[lax.rem(k, RING)], sem_ring)

    for k in range(RING - 1):
        issue(k)

    def flush(fc):
        base = pl.multiple_of(wid * REGION + (fc - FLUSH), FLUSH)
        pltpu.async_copy(rowstage,
                         rows_hbm.at[pl.ds(base, FLUSH), :], sem_st).wait()

    def k_body(k, fcnt):
        slot = lax.rem(k, RING)
        pltpu.make_async_copy(tab_hbm.at[:, pl.ds(0, WIN)], ring.at[slot],
                              sem_ring).wait()
        issue(k + RING - 1)

        c = bk_cnt[pl.ds(k, LANES)][0]
        slotv = jnp.full((LANES,), slot, jnp.int32)

        def j_body(j, fc):
            idj = bk_id[pl.ds(k * CAP + j, LANES)][0]
            bj = bk_b[pl.ds(k * CAP + j, LANES)][0]
            offv = jnp.full((LANES,), idj & (WIN - 1), jnp.int32)
            fcm = fc & (FLUSH - 1)
            for jc in range(EMBED_DIM // LANES):
                row = plsc.load_gather(ring, [slotv, jc * LANES + lane, offv])
                rowstage[fcm, pl.ds(jc * LANES, LANES)] = row
            plsc.store_scatter(blist_v,
                               [jnp.full((LANES,), fc >> 7, jnp.int32),
                                jnp.full((LANES,), fcm, jnp.int32)],
                               jnp.full((LANES,), bj, jnp.int32),
                               mask=lane == 0)
            fc = fc + 1

            @pl.when((fc & (FLUSH - 1)) == 0)
            def _flush_now():
                flush(fc)

            return fc

        return lax.fori_loop(0, c, j_body, fcnt)

    fcnt = lax.fori_loop(0, WPW, k_body, 0)

    @pl.when((fcnt & (FLUSH - 1)) != 0)
    def _flush_tail():
        flush((fcnt & ~(FLUSH - 1)) + FLUSH)

    pltpu.sync_copy(blist_v,
                    bl_hbm.at[pl.ds(wid * (REGION // WIN), REGION // WIN), :])

    for _ in range(RING - 1):
        pltpu.make_async_copy(tab_hbm.at[:, pl.ds(0, WIN)], ring.at[0],
                              sem_ring).wait()


def _sweep(uids_hbm, iids_hbm, ut_hbm, it_hbm,
           urows_hbm, irows_hbm, ubl_hbm, ibl_hbm,
           ids_v, tmp_id, tmp_b, bk_id, bk_b, bk_cnt,
           ring, rowstage, blist_v, sem_ring, sem_st):
    wid = lax.axis_index("s") * NUM_CORES + lax.axis_index("c")
    lane = lax.iota(jnp.int32, LANES)
    args = (wid, lane, ids_v, tmp_id, tmp_b, bk_id, bk_b, bk_cnt,
            ring, rowstage, blist_v, sem_ring, sem_st)
    _one_table(uids_hbm, ut_hbm, urows_hbm, ubl_hbm, *args)
    _one_table(iids_hbm, it_hbm, irows_hbm, ibl_hbm, *args)


def _invert(ubl_hbm, ibl_hbm, posu_hbm, posi_hbm, bl_v, rv_v, sem):
    wid = lax.axis_index("s") * NUM_CORES + lax.axis_index("c")
    lane = lax.iota(jnp.int32, LANES)
    nrows = REGION // WIN
    copies = []
    for t, (bl_hbm, pos_hbm) in enumerate(((ubl_hbm, posu_hbm),
                                           (ibl_hbm, posi_hbm))):
        pltpu.sync_copy(bl_hbm.at[pl.ds(wid * nrows, nrows), :],
                        bl_v.at[pl.ds(t * nrows, nrows), :])
        for r in range(nrows):
            for i in range(0, WIN, LANES):
                rv_v[t * nrows + r, pl.ds(i, LANES)] = (
                    wid * REGION + r * WIN + i + lane)
        for r in range(nrows):
            copies.append(pltpu.async_copy(
                rv_v.at[t * nrows + r], pos_hbm.at[bl_v.at[t * nrows + r]],
                sem))
    for c in copies:
        c.wait()


def _dot(urows_hbm, irows_hbm, posu_hbm, posi_hbm, out_hbm,
         pu_v, pi_v, ur_v, ir_v, out_v, sem):
    wid = lax.axis_index("s") * NUM_CORES + lax.axis_index("c")
    base = wid * B_PER_W
    lane = lax.iota(jnp.int32, LANES)
    nidx = B_PER_W // WIN

    for h in range(nidx):
        pltpu.sync_copy(posu_hbm.at[pl.ds(base + h * WIN, WIN)], pu_v.at[h])
        pltpu.sync_copy(posi_hbm.at[pl.ds(base + h * WIN, WIN)], pi_v.at[h])

    for half in range(2):
        copies = []
        for h in range(2):
            hh = half * 2 + h
            copies.append(pltpu.async_copy(
                urows_hbm.at[pu_v.at[hh]],
                ur_v.at[pl.ds(h * WIN, WIN), :], sem))
            copies.append(pltpu.async_copy(
                irows_hbm.at[pi_v.at[hh]],
                ir_v.at[pl.ds(h * WIN, WIN), :], sem))
        for c in copies:
            c.wait()

        def group_body(g, _):
            b0 = g * LANES
            out16 = jnp.zeros((LANES,), jnp.float32)
            for l in range(LANES):
                b = b0 + l
                acc = ur_v[b, pl.ds(0, LANES)] * ir_v[b, pl.ds(0, LANES)]
                for c in range(1, EMBED_DIM // LANES):
                    acc += (ur_v[b, pl.ds(c * LANES, LANES)]
                            * ir_v[b, pl.ds(c * LANES, LANES)])
                out16 = jnp.where(lane == l, jnp.sum(acc), out16)
            out_v[pl.ds(half * 256 + b0, LANES)] = out16
            return 0

        lax.fori_loop(0, 256 // LANES, group_body, 0)

    pltpu.sync_copy(out_v, out_hbm.at[pl.ds(base, B_PER_W)])


@jax.jit
def _bprmf_score(user_ids, item_ids, user_table, item_table):
    mesh = plsc.VectorSubcoreMesh(core_axis_name="c", subcore_axis_name="s",
                                  num_cores=NUM_CORES,
                                  num_subcores=NUM_SUBCORES)
    tiled = pltpu.CompilerParams(needs_layout_passes=False)
    linear = pltpu.CompilerParams(needs_layout_passes=False,
                                  use_tc_tiling_on_sc=False)
    f32 = jnp.float32
    i32 = jnp.int32

    urows, irows, ubl, ibl = pl.kernel(
        _sweep,
        out_type=(jax.ShapeDtypeStruct((NSTAGE, WIN), f32),
                  jax.ShapeDtypeStruct((NSTAGE, WIN), f32),
                  jax.ShapeDtypeStruct((NSTAGE // WIN, WIN), i32),
                  jax.ShapeDtypeStruct((NSTAGE // WIN, WIN), i32)),
        mesh=mesh,
        compiler_params=tiled,
        scratch_types=[
            pltpu.VMEM((IDBLK,), i32),
            pltpu.VMEM((2 * LANES,), i32),
            pltpu.VMEM((2 * LANES,), i32),
            pltpu.VMEM((WPW * CAP + LANES,), i32),
            pltpu.VMEM((WPW * CAP + LANES,), i32),
            pltpu.VMEM((WPW + 2 * LANES,), i32),
            pltpu.VMEM((RING, EMBED_DIM, WIN), f32),
            pltpu.VMEM((FLUSH, WIN), f32),
            pltpu.VMEM((REGION // WIN, WIN), i32),
            pltpu.SemaphoreType.DMA,
            pltpu.SemaphoreType.DMA,
        ],
    )(user_ids, item_ids, user_table.T, item_table.T)

    posu, posi = pl.kernel(
        _invert,
        out_type=(jax.ShapeDtypeStruct((BATCH + WIN,), i32),
                  jax.ShapeDtypeStruct((BATCH + WIN,), i32)),
        mesh=mesh,
        compiler_params=linear,
        scratch_types=[
            pltpu.VMEM((2 * REGION // WIN, WIN), i32),
            pltpu.VMEM((2 * REGION // WIN, WIN), i32),
            pltpu.SemaphoreType.DMA,
        ],
    )(ubl, ibl)

    return pl.kernel(
        _dot,
        out_type=jax.ShapeDtypeStruct((BATCH,), f32),
        mesh=mesh,
        compiler_params=linear,
        scratch_types=[
            pltpu.VMEM((B_PER_W // WIN, WIN), i32),
            pltpu.VMEM((B_PER_W // WIN, WIN), i32),
            pltpu.VMEM((256, WIN), f32),
            pltpu.VMEM((256, WIN), f32),
            pltpu.VMEM((B_PER_W,), f32),
            pltpu.SemaphoreType.DMA,
        ],
    )(urows, irows, posu, posi)


def kernel(user_ids, item_ids, user_table, item_table):
    return _bprmf_score(user_ids.astype(jnp.int32), item_ids.astype(jnp.int32),
                        user_table, item_table)


# speedup vs baseline: 7.5309x; 7.5309x over previous
import jax
import jax.numpy as jnp
from jax import lax
from jax.experimental import pallas as pl
from jax.experimental.pallas import tpu as pltpu
from jax.experimental.pallas import tpu_sc as plsc

NUM_CORES = 2
NUM_SUBCORES = 16
LANES = 16
NW = NUM_CORES * NUM_SUBCORES

BATCH = 16384
EMBED_DIM = 64
B_PER_W = BATCH // NW
WIN = 128
NWIN_TOT = 7813
WPW = 245
CAP = 16
RING = 6
FLUSH = 128
REGION = 1024
DUMMY = BATCH
IDBLK = 1024
LAST_START = (NWIN_TOT - 1) * WIN
NSTAGE = NW * REGION


def _one_table(ids_hbm, tab_hbm, rows_hbm, wid, lane,
               ids_v, tmp_id, tmp_b, bk_id, bk_b, bk_cnt,
               ring, rowstage, blist_v, sem_ring, sem_st):
    wlo = wid * WPW

    z = jnp.zeros((LANES,), jnp.int32)
    for i in range(0, WPW + LANES, LANES):
        bk_cnt[pl.ds(i, LANES)] = z

    dv = jnp.full((LANES,), DUMMY, jnp.int32)
    for r in range(REGION // WIN):
        for i in range(0, WIN, LANES):
            blist_v[r, pl.ds(i, LANES)] = dv

    def blk_body(blk, _):
        pltpu.sync_copy(ids_hbm.at[pl.ds(blk * IDBLK, IDBLK)], ids_v)

        def v_body(v, _):
            ids16 = ids_v[pl.ds(v * LANES, LANES)]
            m = lax.div(ids16 >> 7, WPW) == wid
            n = plsc.all_reduce_population_count(m)[0]

            @pl.when(n > 0)
            def _binned():
                plsc.store_compressed(tmp_id.at[pl.ds(0, LANES)], ids16,
                                      mask=m)
                b16 = blk * IDBLK + v * LANES + lane
                plsc.store_compressed(tmp_b.at[pl.ds(0, LANES)], b16, mask=m)

                def j_body(j, _):
                    idj = tmp_id[pl.ds(j, LANES)][0]
                    bj = tmp_b[pl.ds(j, LANES)][0]
                    k = (idj >> 7) - wlo
                    c = bk_cnt[pl.ds(k, LANES)][0]
                    sel = lane == 0
                    kc = jnp.full((LANES,), k * CAP + c, jnp.int32)
                    plsc.store_scatter(bk_id, [kc],
                                       jnp.full((LANES,), idj, jnp.int32),
                                       mask=sel)
                    plsc.store_scatter(bk_b, [kc],
                                       jnp.full((LANES,), bj, jnp.int32),
                                       mask=sel)
                    plsc.store_scatter(bk_cnt,
                                       [jnp.full((LANES,), k, jnp.int32)],
                                       jnp.full((LANES,), c + 1, jnp.int32),
                                       mask=sel)
                    return 0

                lax.fori_loop(0, n, j_body, 0)

            return 0

        lax.fori_loop(0, IDBLK // LANES, v_body, 0)
        return 0

    lax.fori_loop(0, BATCH // IDBLK, blk_body, 0)

    def issue(k):
        start = pl.multiple_of(
            jnp.minimum((wlo + k) * WIN, LAST_START), WIN)
        return pltpu.async_copy(tab_hbm.at[:, pl.ds(start, WIN)],
                                ring.at[lax.rem(k, RING)], sem_ring)

    for k in range(RING - 1):
        issue(k)

    def flush(fc):
        r = ((fc - FLUSH) >> 7) & (REGION // WIN - 1)
        pltpu.async_copy(rowstage, rows_hbm.at[blist_v.at[r]], sem_st).wait()

        dv = jnp.full((LANES,), DUMMY, jnp.int32)
        for i in range(0, WIN, LANES):
            blist_v[r, pl.ds(i, LANES)] = dv

    def k_body(k, fcnt):
        slot = lax.rem(k, RING)
        pltpu.make_async_copy(tab_hbm.at[:, pl.ds(0, WIN)], ring.at[slot],
                              sem_ring).wait()
        issue(k + RING - 1)

        c = bk_cnt[pl.ds(k, LANES)][0]
        slotv = jnp.full((LANES,), slot, jnp.int32)

        def j_body(j, fc):
            idj = bk_id[pl.ds(k * CAP + j, LANES)][0]
            bj = bk_b[pl.ds(k * CAP + j, LANES)][0]
            offv = jnp.full((LANES,), idj & (WIN - 1), jnp.int32)
            fcm = fc & (FLUSH - 1)
            for jc in range(EMBED_DIM // LANES):
                row = plsc.load_gather(ring, [slotv, jc * LANES + lane, offv])
                rowstage[fcm, pl.ds(jc * LANES, LANES)] = row
            plsc.store_scatter(blist_v,
                               [jnp.full((LANES,), (fc >> 7)
                                         & (REGION // WIN - 1), jnp.int32),
                                jnp.full((LANES,), fcm, jnp.int32)],
                               jnp.full((LANES,), bj, jnp.int32),
                               mask=lane == 0)
            fc = fc + 1

            @pl.when((fc & (FLUSH - 1)) == 0)
            def _flush_now():
                flush(fc)

            return fc

        return lax.fori_loop(0, c, j_body, fcnt)

    fcnt = lax.fori_loop(0, WPW, k_body, 0)

    @pl.when((fcnt & (FLUSH - 1)) != 0)
    def _flush_tail():
        flush((fcnt & ~(FLUSH - 1)) + FLUSH)

    for _ in range(RING - 1):
        pltpu.make_async_copy(tab_hbm.at[:, pl.ds(0, WIN)], ring.at[0],
                              sem_ring).wait()


def _sweep(uids_hbm, iids_hbm, ut_hbm, it_hbm,
           urows_hbm, irows_hbm,
           ids_v, tmp_id, tmp_b, bk_id, bk_b, bk_cnt,
           ring, rowstage, blist_v, sem_ring, sem_st):
    wid = lax.axis_index("s") * NUM_CORES + lax.axis_index("c")
    lane = lax.iota(jnp.int32, LANES)
    args = (wid, lane, ids_v, tmp_id, tmp_b, bk_id, bk_b, bk_cnt,
            ring, rowstage, blist_v, sem_ring, sem_st)
    _one_table(uids_hbm, ut_hbm, urows_hbm, *args)
    _one_table(iids_hbm, it_hbm, irows_hbm, *args)


def _dot(urows_hbm, irows_hbm, out_hbm,
         ur_v, ir_v, out_v, sem):
    wid = lax.axis_index("s") * NUM_CORES + lax.axis_index("c")
    base = wid * B_PER_W
    lane = lax.iota(jnp.int32, LANES)

    for half in range(2):
        cu = pltpu.async_copy(
            urows_hbm.at[pl.ds(base + half * 256, 256), :], ur_v, sem)
        ci = pltpu.async_copy(
            irows_hbm.at[pl.ds(base + half * 256, 256), :], ir_v, sem)
        cu.wait()
        ci.wait()

        def group_body(g, _):
            b0 = g * LANES
            out16 = jnp.zeros((LANES,), jnp.float32)
            for l in range(LANES):
                b = b0 + l
                acc = ur_v[b, pl.ds(0, LANES)] * ir_v[b, pl.ds(0, LANES)]
                for c in range(1, EMBED_DIM // LANES):
                    acc += (ur_v[b, pl.ds(c * LANES, LANES)]
                            * ir_v[b, pl.ds(c * LANES, LANES)])
                out16 = jnp.where(lane == l, jnp.sum(acc), out16)
            out_v[pl.ds(half * 256 + b0, LANES)] = out16
            return 0

        lax.fori_loop(0, 256 // LANES, group_body, 0)

    pltpu.sync_copy(out_v, out_hbm.at[pl.ds(base, B_PER_W)])


@jax.jit
def _bprmf_score(user_ids, item_ids, user_table, item_table):
    mesh = plsc.VectorSubcoreMesh(core_axis_name="c", subcore_axis_name="s",
                                  num_cores=NUM_CORES,
                                  num_subcores=NUM_SUBCORES)
    tiled = pltpu.CompilerParams(needs_layout_passes=False)
    linear = pltpu.CompilerParams(needs_layout_passes=False,
                                  use_tc_tiling_on_sc=False)
    f32 = jnp.float32
    i32 = jnp.int32

    urows, irows = pl.kernel(
        _sweep,
        out_type=(jax.ShapeDtypeStruct((BATCH + LANES, WIN), f32),
                  jax.ShapeDtypeStruct((BATCH + LANES, WIN), f32)),
        mesh=mesh,
        compiler_params=tiled,
        scratch_types=[
            pltpu.VMEM((IDBLK,), i32),
            pltpu.VMEM((2 * LANES,), i32),
            pltpu.VMEM((2 * LANES,), i32),
            pltpu.VMEM((WPW * CAP + LANES,), i32),
            pltpu.VMEM((WPW * CAP + LANES,), i32),
            pltpu.VMEM((WPW + 2 * LANES,), i32),
            pltpu.VMEM((RING, EMBED_DIM, WIN), f32),
            pltpu.VMEM((FLUSH, WIN), f32),
            pltpu.VMEM((REGION // WIN, WIN), i32),
            pltpu.SemaphoreType.DMA,
            pltpu.SemaphoreType.DMA,
        ],
    )(user_ids, item_ids, user_table.T, item_table.T)

    return pl.kernel(
        _dot,
        out_type=jax.ShapeDtypeStruct((BATCH,), f32),
        mesh=mesh,
        compiler_params=linear,
        scratch_types=[
            pltpu.VMEM((256, WIN), f32),
            pltpu.VMEM((256, WIN), f32),
            pltpu.VMEM((B_PER_W,), f32),
            pltpu.SemaphoreType.DMA,
        ],
    )(urows, irows)


def kernel(user_ids, item_ids, user_table, item_table):
    return _bprmf_score(user_ids.astype(jnp.int32), item_ids.astype(jnp.int32),
                        user_table, item_table)


# speedup vs baseline: 7.8433x; 1.0415x over previous
import jax
import jax.numpy as jnp
from jax import lax
from jax.experimental import pallas as pl
from jax.experimental.pallas import tpu as pltpu
from jax.experimental.pallas import tpu_sc as plsc

NUM_CORES = 2
NUM_SUBCORES = 16
LANES = 16
NW = NUM_CORES * NUM_SUBCORES

BATCH = 16384
EMBED_DIM = 64
B_PER_W = BATCH // NW
WIN = 128
NWIN_TOT = 7813
WPW = 245
CAP = 16
RING = 6
FLUSH = 128
REGION = 1024
DUMMY = BATCH
IDBLK = 1024
LAST_START = (NWIN_TOT - 1) * WIN
NSTAGE = NW * REGION


def _one_table(ids_hbm, tab_hbm, rows_hbm, wid, lane,
               ids_v, tmp_id, tmp_b, bk_id, bk_cnt,
               ring, rowstage, blist_v, sem_ring, sem_st):
    wlo = wid * WPW

    z = jnp.zeros((LANES,), jnp.int32)
    for i in range(0, WPW + LANES, LANES):
        bk_cnt[pl.ds(i, LANES)] = z

    dv = jnp.full((LANES,), DUMMY, jnp.int32)
    for r in range(REGION // WIN):
        for i in range(0, WIN, LANES):
            blist_v[r, pl.ds(i, LANES)] = dv

    def blk_body(blk, _):
        pltpu.sync_copy(ids_hbm.at[pl.ds(blk * IDBLK, IDBLK)], ids_v)

        def v_body(v, _):
            ids16 = ids_v[pl.ds(v * LANES, LANES)]
            m = (((ids16 >> 7) * 4280) >> 20) == wid
            n = plsc.all_reduce_population_count(m)[0]

            @pl.when(n > 0)
            def _binned():
                plsc.store_compressed(tmp_b.at[pl.ds(0, LANES)], ids16, mask=m)
                b16 = blk * IDBLK + v * LANES + lane
                plsc.store_compressed(tmp_b.at[pl.ds(LANES, LANES)], b16,
                                      mask=m)
                ev = tmp_b[pl.ds(0, LANES)]
                od = tmp_b[pl.ds(LANES, LANES)]
                plsc.store_scatter(tmp_id, [2 * lane], ev)
                plsc.store_scatter(tmp_id, [2 * lane + 1], od)

                def j_body(j, _):
                    pair = tmp_id[pl.ds(2 * j, LANES)]
                    idj = pair[0]
                    bj = pair[1]
                    k = (idj >> 7) - wlo
                    c = bk_cnt[pl.ds(k, LANES)][0]
                    kc = jnp.full((LANES,), 2 * (k * CAP + c), jnp.int32)
                    plsc.store_scatter(bk_id, [kc + lane],
                                       jnp.where(lane == 0, idj, bj),
                                       mask=lane < 2)
                    plsc.store_scatter(bk_cnt,
                                       [jnp.full((LANES,), k, jnp.int32)],
                                       jnp.full((LANES,), c + 1, jnp.int32),
                                       mask=lane == 0)
                    return 0

                lax.fori_loop(0, n, j_body, 0)

            return 0

        lax.fori_loop(0, IDBLK // LANES, v_body, 0)
        return 0

    lax.fori_loop(0, BATCH // IDBLK, blk_body, 0)

    def issue(k):
        start = pl.multiple_of(
            jnp.minimum((wlo + k) * WIN, LAST_START), WIN)
        return pltpu.async_copy(tab_hbm.at[:, pl.ds(start, WIN)],
                                ring.at[lax.rem(k, RING)], sem_ring)

    for k in range(RING - 1):
        issue(k)

    def flush(fc):
        r = ((fc - FLUSH) >> 7) & (REGION // WIN - 1)
        pltpu.async_copy(rowstage, rows_hbm.at[blist_v.at[r]], sem_st).wait()

        dv = jnp.full((LANES,), DUMMY, jnp.int32)
        for i in range(0, WIN, LANES):
            blist_v[r, pl.ds(i, LANES)] = dv

    def k_body(k, fcnt):
        slot = lax.rem(k, RING)
        pltpu.make_async_copy(tab_hbm.at[:, pl.ds(0, WIN)], ring.at[slot],
                              sem_ring).wait()
        issue(k + RING - 1)

        c = bk_cnt[pl.ds(k, LANES)][0]
        slotv = jnp.full((LANES,), slot, jnp.int32)

        def j_body(j, fc):
            pair = bk_id[pl.ds(2 * (k * CAP + j), LANES)]
            idj = pair[0]
            bj = pair[1]
            offv = jnp.full((LANES,), idj & (WIN - 1), jnp.int32)
            fcm = fc & (FLUSH - 1)
            for jc in range(EMBED_DIM // LANES):
                row = plsc.load_gather(ring, [slotv, jc * LANES + lane, offv])
                rowstage[fcm, pl.ds(jc * LANES, LANES)] = row
            plsc.store_scatter(blist_v,
                               [jnp.full((LANES,), (fc >> 7)
                                         & (REGION // WIN - 1), jnp.int32),
                                jnp.full((LANES,), fcm, jnp.int32)],
                               jnp.full((LANES,), bj, jnp.int32),
                               mask=lane == 0)
            fc = fc + 1

            @pl.when((fc & (FLUSH - 1)) == 0)
            def _flush_now():
                flush(fc)

            return fc

        return lax.fori_loop(0, c, j_body, fcnt)

    fcnt = lax.fori_loop(0, WPW, k_body, 0)

    @pl.when((fcnt & (FLUSH - 1)) != 0)
    def _flush_tail():
        flush((fcnt & ~(FLUSH - 1)) + FLUSH)

    for _ in range(RING - 1):
        pltpu.make_async_copy(tab_hbm.at[:, pl.ds(0, WIN)], ring.at[0],
                              sem_ring).wait()


def _sweep(uids_hbm, iids_hbm, ut_hbm, it_hbm,
           urows_hbm, irows_hbm,
           ids_v, tmp_id, tmp_b, bk_id, bk_cnt,
           ring, rowstage, blist_v, sem_ring, sem_st):
    wid = lax.axis_index("s") * NUM_CORES + lax.axis_index("c")
    lane = lax.iota(jnp.int32, LANES)
    args = (wid, lane, ids_v, tmp_id, tmp_b, bk_id, bk_cnt,
            ring, rowstage, blist_v, sem_ring, sem_st)
    _one_table(uids_hbm, ut_hbm, urows_hbm, *args)
    _one_table(iids_hbm, it_hbm, irows_hbm, *args)


def _dot(urows_hbm, irows_hbm, out_hbm,
         ur_v, ir_v, out_v, sem):
    wid = lax.axis_index("s") * NUM_CORES + lax.axis_index("c")
    base = wid * B_PER_W
    lane = lax.iota(jnp.int32, LANES)

    for half in range(2):
        cu = pltpu.async_copy(
            urows_hbm.at[pl.ds(base + half * 256, 256), :], ur_v, sem)
        ci = pltpu.async_copy(
            irows_hbm.at[pl.ds(base + half * 256, 256), :], ir_v, sem)
        cu.wait()
        ci.wait()

        def group_body(g, _):
            b0 = g * LANES
            out16 = jnp.zeros((LANES,), jnp.float32)
            for l in range(LANES):
                b = b0 + l
                acc = ur_v[b, pl.ds(0, LANES)] * ir_v[b, pl.ds(0, LANES)]
                for c in range(1, EMBED_DIM // LANES):
                    acc += (ur_v[b, pl.ds(c * LANES, LANES)]
                            * ir_v[b, pl.ds(c * LANES, LANES)])
                out16 = jnp.where(lane == l, jnp.sum(acc), out16)
            out_v[pl.ds(half * 256 + b0, LANES)] = out16
            return 0

        lax.fori_loop(0, 256 // LANES, group_body, 0)

    pltpu.sync_copy(out_v, out_hbm.at[pl.ds(base, B_PER_W)])


@jax.jit
def _bprmf_score(user_ids, item_ids, user_table, item_table):
    mesh = plsc.VectorSubcoreMesh(core_axis_name="c", subcore_axis_name="s",
                                  num_cores=NUM_CORES,
                                  num_subcores=NUM_SUBCORES)
    tiled = pltpu.CompilerParams(needs_layout_passes=False)
    linear = pltpu.CompilerParams(needs_layout_passes=False,
                                  use_tc_tiling_on_sc=False)
    f32 = jnp.float32
    i32 = jnp.int32

    urows, irows = pl.kernel(
        _sweep,
        out_type=(jax.ShapeDtypeStruct((BATCH + LANES, WIN), f32),
                  jax.ShapeDtypeStruct((BATCH + LANES, WIN), f32)),
        mesh=mesh,
        compiler_params=tiled,
        scratch_types=[
            pltpu.VMEM((IDBLK,), i32),
            pltpu.VMEM((4 * LANES,), i32),
            pltpu.VMEM((4 * LANES,), i32),
            pltpu.VMEM((2 * WPW * CAP + LANES,), i32),
            pltpu.VMEM((WPW + 2 * LANES,), i32),
            pltpu.VMEM((RING, EMBED_DIM, WIN), f32),
            pltpu.VMEM((FLUSH, WIN), f32),
            pltpu.VMEM((REGION // WIN, WIN), i32),
            pltpu.SemaphoreType.DMA,
            pltpu.SemaphoreType.DMA,
        ],
    )(user_ids, item_ids, user_table.T, item_table.T)

    return pl.kernel(
        _dot,
        out_type=jax.ShapeDtypeStruct((BATCH,), f32),
        mesh=mesh,
        compiler_params=linear,
        scratch_types=[
            pltpu.VMEM((256, WIN), f32),
            pltpu.VMEM((256, WIN), f32),
            pltpu.VMEM((B_PER_W,), f32),
            pltpu.SemaphoreType.DMA,
        ],
    )(urows, irows)


def kernel(user_ids, item_ids, user_table, item_table):
    return _bprmf_score(user_ids.astype(jnp.int32), item_ids.astype(jnp.int32),
                        user_table, item_table)


# speedup vs baseline: 7.9373x; 1.0120x over previous
import jax
import jax.numpy as jnp
from jax import lax
from jax.experimental import pallas as pl
from jax.experimental.pallas import tpu as pltpu
from jax.experimental.pallas import tpu_sc as plsc

NUM_CORES = 2
NUM_SUBCORES = 16
LANES = 16
NW = NUM_CORES * NUM_SUBCORES

BATCH = 16384
EMBED_DIM = 64
B_PER_W = BATCH // NW
WIN = 128
NWIN_TOT = 7813
WPW = 245
CAP = 16
RING = 6
FLUSH = 128
REGION = 1024
DUMMY = BATCH
IDBLK = 1024
LAST_START = (NWIN_TOT - 1) * WIN
NSTAGE = NW * REGION


def _one_table(ids_hbm, tab_hbm, rows_hbm, wid, lane,
               ids_v, tmp_id, tmp_b, bk_id, bk_cnt,
               ring, rowstage, blist_v, sem_ring, sem_st):
    wlo = wid * WPW

    z = jnp.zeros((LANES,), jnp.int32)
    for i in range(0, WPW + LANES, LANES):
        bk_cnt[pl.ds(i, LANES)] = z

    dv = jnp.full((LANES,), DUMMY, jnp.int32)
    for r in range(REGION // WIN):
        for i in range(0, WIN, LANES):
            blist_v[r, pl.ds(i, LANES)] = dv

    def blk_body(blk, _):
        pltpu.sync_copy(ids_hbm.at[pl.ds(blk * IDBLK, IDBLK)], ids_v)

        def v_body(v, _):
            ids16 = ids_v[pl.ds(v * LANES, LANES)]
            m = (((ids16 >> 7) * 4280) >> 20) == wid
            n = plsc.all_reduce_population_count(m)[0]

            @pl.when(n > 0)
            def _binned():
                plsc.store_compressed(tmp_b.at[pl.ds(0, LANES)], ids16, mask=m)
                b16 = blk * IDBLK + v * LANES + lane
                plsc.store_compressed(tmp_b.at[pl.ds(LANES, LANES)], b16,
                                      mask=m)
                ev = tmp_b[pl.ds(0, LANES)]
                od = tmp_b[pl.ds(LANES, LANES)]
                plsc.store_scatter(tmp_id, [2 * lane], ev)
                plsc.store_scatter(tmp_id, [2 * lane + 1], od)

                def j_body(j, _):
                    pair = tmp_id[pl.ds(2 * j, LANES)]
                    idj = pair[0]
                    bj = pair[1]
                    k = (idj >> 7) - wlo
                    c = bk_cnt[pl.ds(k, LANES)][0]
                    kc = jnp.full((LANES,), 2 * (k * CAP + c), jnp.int32)
                    plsc.store_scatter(bk_id, [kc + lane],
                                       jnp.where(lane == 0, idj, bj),
                                       mask=lane < 2)
                    plsc.store_scatter(bk_cnt,
                                       [jnp.full((LANES,), k, jnp.int32)],
                                       jnp.full((LANES,), c + 1, jnp.int32),
                                       mask=lane == 0)
                    return 0

                lax.fori_loop(0, n, j_body, 0)

            return 0

        lax.fori_loop(0, IDBLK // LANES, v_body, 0, unroll=4)
        return 0

    lax.fori_loop(0, BATCH // IDBLK, blk_body, 0)

    def issue(k):
        start = pl.multiple_of(
            jnp.minimum((wlo + k) * WIN, LAST_START), WIN)
        return pltpu.async_copy(tab_hbm.at[:, pl.ds(start, WIN)],
                                ring.at[lax.rem(k, RING)], sem_ring)

    for k in range(RING - 1):
        issue(k)

    def flush(fc):
        r = ((fc - FLUSH) >> 7) & (REGION // WIN - 1)
        pltpu.async_copy(rowstage, rows_hbm.at[blist_v.at[r]], sem_st).wait()

        dv = jnp.full((LANES,), DUMMY, jnp.int32)
        for i in range(0, WIN, LANES):
            blist_v[r, pl.ds(i, LANES)] = dv

    def k_body(k, fcnt):
        slot = lax.rem(k, RING)
        pltpu.make_async_copy(tab_hbm.at[:, pl.ds(0, WIN)], ring.at[slot],
                              sem_ring).wait()
        issue(k + RING - 1)

        c = bk_cnt[pl.ds(k, LANES)][0]
        slotv = jnp.full((LANES,), slot, jnp.int32)

        def j_body(j, fc):
            pair = bk_id[pl.ds(2 * (k * CAP + j), LANES)]
            idj = pair[0]
            bj = pair[1]
            offv = jnp.full((LANES,), idj & (WIN - 1), jnp.int32)
            fcm = fc & (FLUSH - 1)
            for jc in range(EMBED_DIM // LANES):
                row = plsc.load_gather(ring, [slotv, jc * LANES + lane, offv])
                rowstage[fcm, pl.ds(jc * LANES, LANES)] = row
            plsc.store_scatter(blist_v,
                               [jnp.full((LANES,), (fc >> 7)
                                         & (REGION // WIN - 1), jnp.int32),
                                jnp.full((LANES,), fcm, jnp.int32)],
                               jnp.full((LANES,), bj, jnp.int32),
                               mask=lane == 0)
            fc = fc + 1

            @pl.when((fc & (FLUSH - 1)) == 0)
            def _flush_now():
                flush(fc)

            return fc

        return lax.fori_loop(0, c, j_body, fcnt)

    fcnt = lax.fori_loop(0, WPW, k_body, 0)

    @pl.when((fcnt & (FLUSH - 1)) != 0)
    def _flush_tail():
        flush((fcnt & ~(FLUSH - 1)) + FLUSH)

    for _ in range(RING - 1):
        pltpu.make_async_copy(tab_hbm.at[:, pl.ds(0, WIN)], ring.at[0],
                              sem_ring).wait()


def _sweep(uids_hbm, iids_hbm, ut_hbm, it_hbm,
           urows_hbm, irows_hbm,
           ids_v, tmp_id, tmp_b, bk_id, bk_cnt,
           ring, rowstage, blist_v, sem_ring, sem_st):
    wid = lax.axis_index("s") * NUM_CORES + lax.axis_index("c")
    lane = lax.iota(jnp.int32, LANES)
    args = (wid, lane, ids_v, tmp_id, tmp_b, bk_id, bk_cnt,
            ring, rowstage, blist_v, sem_ring, sem_st)
    _one_table(uids_hbm, ut_hbm, urows_hbm, *args)
    _one_table(iids_hbm, it_hbm, irows_hbm, *args)


def _dot(urows_hbm, irows_hbm, out_hbm,
         ur_v, ir_v, out_v, sem):
    wid = lax.axis_index("s") * NUM_CORES + lax.axis_index("c")
    base = wid * B_PER_W
    lane = lax.iota(jnp.int32, LANES)

    for half in range(2):
        cu = pltpu.async_copy(
            urows_hbm.at[pl.ds(base + half * 256, 256), :], ur_v, sem)
        ci = pltpu.async_copy(
            irows_hbm.at[pl.ds(base + half * 256, 256), :], ir_v, sem)
        cu.wait()
        ci.wait()

        def group_body(g, _):
            b0 = g * LANES
            out16 = jnp.zeros((LANES,), jnp.float32)
            for l in range(LANES):
                b = b0 + l
                acc = ur_v[b, pl.ds(0, LANES)] * ir_v[b, pl.ds(0, LANES)]
                for c in range(1, EMBED_DIM // LANES):
                    acc += (ur_v[b, pl.ds(c * LANES, LANES)]
                            * ir_v[b, pl.ds(c * LANES, LANES)])
                out16 = jnp.where(lane == l, jnp.sum(acc), out16)
            out_v[pl.ds(half * 256 + b0, LANES)] = out16
            return 0

        lax.fori_loop(0, 256 // LANES, group_body, 0)

    pltpu.sync_copy(out_v, out_hbm.at[pl.ds(base, B_PER_W)])


@jax.jit
def _bprmf_score(user_ids, item_ids, user_table, item_table):
    mesh = plsc.VectorSubcoreMesh(core_axis_name="c", subcore_axis_name="s",
                                  num_cores=NUM_CORES,
                                  num_subcores=NUM_SUBCORES)
    tiled = pltpu.CompilerParams(needs_layout_passes=False)
    linear = pltpu.CompilerParams(needs_layout_passes=False,
                                  use_tc_tiling_on_sc=False)
    f32 = jnp.float32
    i32 = jnp.int32

    urows, irows = pl.kernel(
        _sweep,
        out_type=(jax.ShapeDtypeStruct((BATCH + LANES, WIN), f32),
                  jax.ShapeDtypeStruct((BATCH + LANES, WIN), f32)),
        mesh=mesh,
        compiler_params=tiled,
        scratch_types=[
            pltpu.VMEM((IDBLK,), i32),
            pltpu.VMEM((4 * LANES,), i32),
            pltpu.VMEM((4 * LANES,), i32),
            pltpu.VMEM((2 * WPW * CAP + LANES,), i32),
            pltpu.VMEM((WPW + 2 * LANES,), i32),
            pltpu.VMEM((RING, EMBED_DIM, WIN), f32),
            pltpu.VMEM((FLUSH, WIN), f32),
            pltpu.VMEM((REGION // WIN, WIN), i32),
            pltpu.SemaphoreType.DMA,
            pltpu.SemaphoreType.DMA,
        ],
    )(user_ids, item_ids, user_table.T, item_table.T)

    return pl.kernel(
        _dot,
        out_type=jax.ShapeDtypeStruct((BATCH,), f32),
        mesh=mesh,
        compiler_params=linear,
        scratch_types=[
            pltpu.VMEM((256, WIN), f32),
            pltpu.VMEM((256, WIN), f32),
            pltpu.VMEM((B_PER_W,), f32),
            pltpu.SemaphoreType.DMA,
        ],
    )(urows, irows)


def kernel(user_ids, item_ids, user_table, item_table):
    return _bprmf_score(user_ids.astype(jnp.int32), item_ids.astype(jnp.int32),
                        user_table, item_table)


# speedup vs baseline: 10.3839x; 1.3083x over previous
import jax
import jax.numpy as jnp
from jax import lax
from jax.experimental import pallas as pl
from jax.experimental.pallas import tpu as pltpu
from jax.experimental.pallas import tpu_sc as plsc

NUM_CORES = 2
NUM_SUBCORES = 16
LANES = 16
NW = NUM_CORES * NUM_SUBCORES

BATCH = 16384
EMBED_DIM = 64
B_PER_W = BATCH // NW
WIN = 128
NBUF = 7


def _window_copy(tab_hbm, bufs, slot, col, sem):
    start = pl.multiple_of((col >> 7) << 7, WIN)
    return pltpu.async_copy(tab_hbm.at[:, pl.ds(start, WIN)], bufs.at[slot],
                            sem)


def _body(user_ids_hbm, item_ids_hbm, ut_hbm, it_hbm, out_hbm,
          uidx_v, iidx_v, ubufs, ibufs, out_v, sem_u, sem_i):
    wid = lax.axis_index("s") * NUM_CORES + lax.axis_index("c")
    base = wid * B_PER_W

    pltpu.sync_copy(user_ids_hbm.at[pl.ds(base, B_PER_W)],
                    uidx_v.at[pl.ds(0, B_PER_W)])
    pltpu.sync_copy(item_ids_hbm.at[pl.ds(base, B_PER_W)],
                    iidx_v.at[pl.ds(0, B_PER_W)])

    lane = lax.iota(jnp.int32, LANES)

    def ids_at(b):
        return uidx_v[pl.ds(b, LANES)][0], iidx_v[pl.ds(b, LANES)][0]

    for b in range(NBUF - 1):
        uid, iid = ids_at(b)
        _window_copy(ut_hbm, ubufs, b, uid, sem_u)
        _window_copy(it_hbm, ibufs, b, iid, sem_i)

    def b_body(b, out16):
        p = lax.rem(b, NBUF)
        pltpu.make_async_copy(ut_hbm.at[:, pl.ds(0, WIN)], ubufs.at[p],
                              sem_u).wait()
        pltpu.make_async_copy(it_hbm.at[:, pl.ds(0, WIN)], ibufs.at[p],
                              sem_i).wait()

        bn = b + NBUF - 1

        @pl.when(bn < B_PER_W)
        def _():
            uid_n, iid_n = ids_at(bn)
            pn = lax.rem(bn, NBUF)
            _window_copy(ut_hbm, ubufs, pn, uid_n, sem_u)
            _window_copy(it_hbm, ibufs, pn, iid_n, sem_i)

        uid, iid = ids_at(b)
        uoff = jnp.full((LANES,), uid & (WIN - 1), jnp.int32)
        ioff = jnp.full((LANES,), iid & (WIN - 1), jnp.int32)
        pv = jnp.full((LANES,), p, jnp.int32)
        acc = jnp.zeros((LANES,), jnp.float32)
        for jc in range(EMBED_DIM // LANES):
            jrow = jc * LANES + lane
            u = plsc.load_gather(ubufs, [pv, jrow, uoff])
            iv = plsc.load_gather(ibufs, [pv, jrow, ioff])
            acc += u * iv

        l = b & (LANES - 1)
        out16 = jnp.where(l == 0, jnp.zeros((LANES,), jnp.float32), out16)
        out16 = jnp.where(lane == l, jnp.sum(acc), out16)

        @pl.when(l == LANES - 1)
        def _():
            out_v[pl.ds(b - (LANES - 1), LANES)] = out16

        return out16

    lax.fori_loop(0, B_PER_W, b_body, jnp.zeros((LANES,), jnp.float32))

    pltpu.sync_copy(out_v, out_hbm.at[pl.ds(base, B_PER_W)])


@jax.jit
def _bprmf_score(user_ids, item_ids, user_table, item_table):
    mesh = plsc.VectorSubcoreMesh(core_axis_name="c", subcore_axis_name="s",
                                  num_cores=NUM_CORES,
                                  num_subcores=NUM_SUBCORES)
    return pl.kernel(
        _body,
        out_type=jax.ShapeDtypeStruct((BATCH,), jnp.float32),
        mesh=mesh,
        compiler_params=pltpu.CompilerParams(needs_layout_passes=False),
        scratch_types=[
            pltpu.VMEM((B_PER_W + LANES,), jnp.int32),
            pltpu.VMEM((B_PER_W + LANES,), jnp.int32),
            pltpu.VMEM((NBUF, EMBED_DIM, WIN), jnp.float32),
            pltpu.VMEM((NBUF, EMBED_DIM, WIN), jnp.float32),
            pltpu.VMEM((B_PER_W,), jnp.float32),
            pltpu.SemaphoreType.DMA,
            pltpu.SemaphoreType.DMA,
        ],
    )(user_ids, item_ids, user_table.T, item_table.T)


def kernel(user_ids, item_ids, user_table, item_table):
    return _bprmf_score(user_ids.astype(jnp.int32), item_ids.astype(jnp.int32),
                        user_table, item_table)


# speedup vs baseline: 10.5276x; 1.0138x over previous
import jax
import jax.numpy as jnp
from jax import lax
from jax.experimental import pallas as pl
from jax.experimental.pallas import tpu as pltpu
from jax.experimental.pallas import tpu_sc as plsc

NUM_CORES = 2
NUM_SUBCORES = 16
LANES = 16
NW = NUM_CORES * NUM_SUBCORES

BATCH = 16384
EMBED_DIM = 64
B_PER_W = BATCH // NW
WIN = 128
NBUF = 6


def _window_copy(tab_hbm, bufs, slot, col, sem):
    start = pl.multiple_of((col >> 7) << 7, WIN)
    return pltpu.async_copy(tab_hbm.at[:, pl.ds(start, WIN)], bufs.at[slot],
                            sem)


def _body(user_ids_hbm, item_ids_hbm, ut_hbm, it_hbm, out_hbm,
          uidx_v, iidx_v, ubufs, ibufs, out_v, sem_u, sem_i):
    wid = lax.axis_index("s") * NUM_CORES + lax.axis_index("c")
    base = wid * B_PER_W

    pltpu.sync_copy(user_ids_hbm.at[pl.ds(base, B_PER_W)],
                    uidx_v.at[pl.ds(0, B_PER_W)])
    pltpu.sync_copy(item_ids_hbm.at[pl.ds(base, B_PER_W)],
                    iidx_v.at[pl.ds(0, B_PER_W)])

    lane = lax.iota(jnp.int32, LANES)

    def ids_at(b):
        return uidx_v[pl.ds(b, LANES)][0], iidx_v[pl.ds(b, LANES)][0]

    for b in range(NBUF - 1):
        uid, iid = ids_at(b)
        _window_copy(ut_hbm, ubufs, b, uid, sem_u)
        _window_copy(it_hbm, ibufs, b, iid, sem_i)

    def b_body(b, out16):
        p = lax.rem(b, NBUF)
        pltpu.make_async_copy(ut_hbm.at[:, pl.ds(0, WIN)], ubufs.at[p],
                              sem_u).wait()
        pltpu.make_async_copy(it_hbm.at[:, pl.ds(0, WIN)], ibufs.at[p],
                              sem_i).wait()

        bn = b + NBUF - 1

        @pl.when(bn < B_PER_W)
        def _():
            uid_n, iid_n = ids_at(bn)
            pn = lax.rem(bn, NBUF)
            _window_copy(ut_hbm, ubufs, pn, uid_n, sem_u)
            _window_copy(it_hbm, ibufs, pn, iid_n, sem_i)

        uid, iid = ids_at(b)
        uoff = jnp.full((LANES,), uid & (WIN - 1), jnp.int32)
        ioff = jnp.full((LANES,), iid & (WIN - 1), jnp.int32)
        pv = jnp.full((LANES,), p, jnp.int32)
        acc = jnp.zeros((LANES,), jnp.float32)
        for jc in range(EMBED_DIM // LANES):
            jrow = jc * LANES + lane
            u = plsc.load_gather(ubufs, [pv, jrow, uoff])
            iv = plsc.load_gather(ibufs, [pv, jrow, ioff])
            acc += u * iv

        l = b & (LANES - 1)
        out16 = jnp.where(l == 0, jnp.zeros((LANES,), jnp.float32), out16)
        out16 = jnp.where(lane == l, jnp.sum(acc), out16)

        @pl.when(l == LANES - 1)
        def _():
            out_v[pl.ds(b - (LANES - 1), LANES)] = out16

        return out16

    lax.fori_loop(0, B_PER_W, b_body, jnp.zeros((LANES,), jnp.float32))

    pltpu.sync_copy(out_v, out_hbm.at[pl.ds(base, B_PER_W)])


@jax.jit
def _bprmf_score(user_ids, item_ids, user_table, item_table):
    mesh = plsc.VectorSubcoreMesh(core_axis_name="c", subcore_axis_name="s",
                                  num_cores=NUM_CORES,
                                  num_subcores=NUM_SUBCORES)
    return pl.kernel(
        _body,
        out_type=jax.ShapeDtypeStruct((BATCH,), jnp.float32),
        mesh=mesh,
        compiler_params=pltpu.CompilerParams(needs_layout_passes=False),
        scratch_types=[
            pltpu.VMEM((B_PER_W + LANES,), jnp.int32),
            pltpu.VMEM((B_PER_W + LANES,), jnp.int32),
            pltpu.VMEM((NBUF, EMBED_DIM, WIN), jnp.float32),
            pltpu.VMEM((NBUF, EMBED_DIM, WIN), jnp.float32),
            pltpu.VMEM((B_PER_W,), jnp.float32),
            pltpu.SemaphoreType.DMA,
            pltpu.SemaphoreType.DMA,
        ],
    )(user_ids, item_ids, user_table.T, item_table.T)


def kernel(user_ids, item_ids, user_table, item_table):
    return _bprmf_score(user_ids.astype(jnp.int32), item_ids.astype(jnp.int32),
                        user_table, item_table)
